# Initial kernel scaffold; baseline (speedup 1.0000x reference)
import jax
import jax.numpy as jnp
from jax.experimental import pallas as pl


def kernel(x, router_W, router_b, W1, b1, W2, b2):
    raise NotImplementedError("write your pallas kernel here")


# speedup vs baseline: 2.9760x; 2.9760x over previous
import jax
import jax.numpy as jnp
from jax import lax
from jax.experimental import pallas as pl
from jax.experimental.pallas import tpu as pltpu
from jax.experimental.pallas import tpu_sc as plsc

N_TOK = 2048
C_DIM = 1024
N_EXP = 8
F_DIM = 4096
TOPK = 2
TILE = 256
MAXT = 23
PROWS = MAXT * TILE
FBLK = 1024
NFB = F_DIM // FBLK
CH = 64


def _router_body(x_ref, w_ref, b_ref, pos_ref, wts_ref, meta_ref, bal_ref):
    f32 = jnp.float32
    xv = x_ref[...]
    logits = jnp.dot(xv, w_ref[...], preferred_element_type=f32) + b_ref[...]
    m = jnp.max(logits, axis=1, keepdims=True)
    ex = jnp.exp(logits - m)
    probs = ex / jnp.sum(ex, axis=1, keepdims=True)
    mean_p = jnp.sum(probs, axis=0, keepdims=True) * (1.0 / N_TOK)
    bal_ref[...] = jnp.sum(mean_p * mean_p, axis=1, keepdims=True)
    lane = lax.broadcasted_iota(jnp.int32, (N_TOK, N_EXP), 1)
    i1 = jnp.min(jnp.where(logits == m, lane, N_EXP), axis=1, keepdims=True)
    masked = jnp.where(lane == i1, -jnp.inf, logits)
    m2 = jnp.max(masked, axis=1, keepdims=True)
    i2 = jnp.min(jnp.where(masked == m2, lane, N_EXP), axis=1, keepdims=True)
    v1 = jnp.sum(jnp.where(lane == i1, probs, 0.0), axis=1, keepdims=True)
    v2 = jnp.sum(jnp.where(lane == i2, probs, 0.0), axis=1, keepdims=True)
    wts_ref[:, 0:1] = v1
    wts_ref[:, 1:2] = v2
    I1 = (lane == i1).astype(f32)
    I2 = (lane == i2).astype(f32)
    counts = jnp.sum(I1 + I2, axis=0, keepdims=True)
    nt = jnp.floor((counts + (TILE - 1)) * (1.0 / TILE))
    er = lax.broadcasted_iota(jnp.int32, (N_EXP, N_EXP), 0)
    ec = lax.broadcasted_iota(jnp.int32, (N_EXP, N_EXP), 1)
    strict = (er < ec).astype(f32)
    tiles_before = jnp.dot(nt, strict, preferred_element_type=f32)
    start = tiles_before * float(TILE)
    tiles_incl = tiles_before + nt
    ident = (er == ec).astype(f32)
    ti_col = jnp.sum(ident * tiles_incl, axis=1, keepdims=True)
    lane32 = lax.broadcasted_iota(jnp.int32, (1, 32), 1).astype(f32)
    te = jnp.sum((ti_col <= lane32).astype(f32), axis=0, keepdims=True)
    te = jnp.minimum(te, float(N_EXP - 1))
    used = tiles_incl[:, N_EXP - 1:N_EXP]
    meta_ref[...] = jnp.where(lane32 == 31.0, used, te).astype(jnp.int32)
    L = (lax.broadcasted_iota(jnp.int32, (128, 128), 0)
         >= lax.broadcasted_iota(jnp.int32, (128, 128), 1)).astype(f32)
    lane8 = lax.broadcasted_iota(jnp.int32, (128, N_EXP), 1)
    carry = jnp.zeros((1, N_EXP), f32)
    for blk in range(32):
        k, rb = divmod(blk, 16)
        r0 = rb * 128
        ei = (i1 if k == 0 else i2)[r0:r0 + 128, :]
        Ic = (lane8 == ei).astype(f32)
        incl = jnp.dot(L, Ic, preferred_element_type=f32)
        rank = carry + incl - Ic
        posb = jnp.sum(Ic * (start + rank), axis=1, keepdims=True)
        pos_ref[r0:r0 + 128, k:k + 1] = posb.astype(jnp.int32)
        carry = carry + incl[127:128, :]


def _router_call(x_flat, router_W, router_b):
    return pl.pallas_call(
        _router_body,
        out_shape=(
            jax.ShapeDtypeStruct((N_TOK, TOPK), jnp.int32),
            jax.ShapeDtypeStruct((N_TOK, TOPK), jnp.float32),
            jax.ShapeDtypeStruct((1, 32), jnp.int32),
            jax.ShapeDtypeStruct((1, 1), jnp.float32),
        ),
    )(x_flat, router_W, router_b.reshape(1, N_EXP))


def _sc_scatter_body(x_hbm, pos_hbm, xg_hbm, xbuf, idxbuf, sem):
    wid = lax.axis_index("s") * 2 + lax.axis_index("c")
    k = wid // 16
    blk = wid % 16
    for sub in range(2):
        tb = blk * 128 + sub * CH
        pltpu.sync_copy(x_hbm.at[pl.ds(tb, CH)], xbuf)
        pltpu.sync_copy(pos_hbm.at[pl.ds(k * N_TOK + tb, CH)], idxbuf)
        pltpu.async_copy(xbuf, xg_hbm.at[idxbuf], sem).wait()


def _sc_scatter(x_flat, pos_flat):
    mesh = plsc.VectorSubcoreMesh(core_axis_name="c", subcore_axis_name="s")
    fn = pl.kernel(
        _sc_scatter_body, mesh=mesh,
        out_type=jax.ShapeDtypeStruct((PROWS, C_DIM), jnp.float32),
        scratch_types=[pltpu.VMEM((CH, C_DIM), jnp.float32),
                       pltpu.VMEM((CH,), jnp.int32),
                       pltpu.SemaphoreType.DMA],
    )
    return fn(x_flat, pos_flat)


def _ffn_body(meta_ref, xg_ref, w1_ref, b1_ref, w2_ref, b2_ref, out_ref):
    t = pl.program_id(0)
    f = pl.program_id(1)
    used = meta_ref[31]

    @pl.when(t < used)
    def _():
        h = jnp.dot(xg_ref[...], w1_ref[0],
                    preferred_element_type=jnp.float32) + b1_ref[0]
        h = 0.5 * h * (1.0 + lax.erf(h * 0.7071067811865476))
        contrib = jnp.dot(h, w2_ref[0], preferred_element_type=jnp.float32)

        @pl.when(f == 0)
        def _():
            out_ref[...] = contrib + b2_ref[0]

        @pl.when(f > 0)
        def _():
            out_ref[...] = out_ref[...] + contrib


def _ffn_call(meta, xg, W1, b1, W2, b2):
    grid_spec = pltpu.PrefetchScalarGridSpec(
        num_scalar_prefetch=1,
        grid=(MAXT, NFB),
        in_specs=[
            pl.BlockSpec((TILE, C_DIM), lambda t, f, m: (t, 0)),
            pl.BlockSpec((1, C_DIM, FBLK), lambda t, f, m: (m[t], 0, f)),
            pl.BlockSpec((1, 1, FBLK), lambda t, f, m: (m[t], 0, f)),
            pl.BlockSpec((1, FBLK, C_DIM), lambda t, f, m: (m[t], f, 0)),
            pl.BlockSpec((1, 1, C_DIM), lambda t, f, m: (m[t], 0, 0)),
        ],
        out_specs=pl.BlockSpec((TILE, C_DIM), lambda t, f, m: (t, 0)),
    )
    return pl.pallas_call(
        _ffn_body, grid_spec=grid_spec,
        out_shape=jax.ShapeDtypeStruct((PROWS, C_DIM), jnp.float32),
    )(meta, xg, W1, b1, W2, b2)


def _sc_gather_body(o_hbm, pos_hbm, g0_hbm, g1_hbm, rbuf, idxbuf, sem):
    wid = lax.axis_index("s") * 2 + lax.axis_index("c")
    tb = wid * CH
    for k in range(2):
        pltpu.sync_copy(pos_hbm.at[pl.ds(k * N_TOK + tb, CH)], idxbuf)
        pltpu.async_copy(o_hbm.at[idxbuf], rbuf, sem).wait()
        g = g0_hbm if k == 0 else g1_hbm
        pltpu.sync_copy(rbuf, g.at[pl.ds(tb, CH)])


def _sc_gather(o, pos_flat):
    mesh = plsc.VectorSubcoreMesh(core_axis_name="c", subcore_axis_name="s")
    fn = pl.kernel(
        _sc_gather_body, mesh=mesh,
        out_type=(jax.ShapeDtypeStruct((N_TOK, C_DIM), jnp.float32),
                  jax.ShapeDtypeStruct((N_TOK, C_DIM), jnp.float32)),
        scratch_types=[pltpu.VMEM((CH, C_DIM), jnp.float32),
                       pltpu.VMEM((CH,), jnp.int32),
                       pltpu.SemaphoreType.DMA],
    )
    return fn(o, pos_flat)


def _combine_body(g0_ref, g1_ref, w_ref, out_ref):
    w = w_ref[...]
    out_ref[...] = g0_ref[...] * w[:, 0:1] + g1_ref[...] * w[:, 1:2]


def _combine_call(g0, g1, wts):
    RB = 512
    return pl.pallas_call(
        _combine_body,
        grid=(N_TOK // RB,),
        in_specs=[pl.BlockSpec((RB, C_DIM), lambda i: (i, 0)),
                  pl.BlockSpec((RB, C_DIM), lambda i: (i, 0)),
                  pl.BlockSpec((RB, TOPK), lambda i: (i, 0))],
        out_specs=pl.BlockSpec((RB, C_DIM), lambda i: (i, 0)),
        out_shape=jax.ShapeDtypeStruct((N_TOK, C_DIM), jnp.float32),
    )(g0, g1, wts)


def kernel(x, router_W, router_b, W1, b1, W2, b2):
    B, T, C = x.shape
    x_flat = x.reshape(T, C)
    pos, wts, meta, bal = _router_call(x_flat, router_W, router_b)
    pos_flat = pos.T.reshape(TOPK * N_TOK)
    xg = _sc_scatter(x_flat, pos_flat)
    o = _ffn_call(meta.reshape(32), xg,
                  W1, b1.reshape(N_EXP, 1, F_DIM),
                  W2, b2.reshape(N_EXP, 1, C_DIM))
    g0, g1 = _sc_gather(o, pos_flat)
    out = _combine_call(g0, g1, wts)
    return out.reshape(B, T, C), bal.reshape(())


# speedup vs baseline: 3.0589x; 1.0279x over previous
import jax
import jax.numpy as jnp
from jax import lax
from jax.experimental import pallas as pl
from jax.experimental.pallas import tpu as pltpu
from jax.experimental.pallas import tpu_sc as plsc

N_TOK = 2048
C_DIM = 1024
N_EXP = 8
F_DIM = 4096
TOPK = 2
TILE = 256
MAXT = 23
PROWS = MAXT * TILE
FBLK = 1024
NFB = F_DIM // FBLK
CH = 64


def _router_body(x_ref, w_ref, b_ref, pos_ref, wts_ref, meta_ref, bal_ref):
    f32 = jnp.float32
    xv = x_ref[...]
    logits = jnp.dot(xv, w_ref[...], preferred_element_type=f32) + b_ref[...]
    m = jnp.max(logits, axis=1, keepdims=True)
    ex = jnp.exp(logits - m)
    probs = ex / jnp.sum(ex, axis=1, keepdims=True)
    mean_p = jnp.sum(probs, axis=0, keepdims=True) * (1.0 / N_TOK)
    bal_ref[...] = jnp.sum(mean_p * mean_p, axis=1, keepdims=True)
    lane = lax.broadcasted_iota(jnp.int32, (N_TOK, N_EXP), 1)
    i1 = jnp.min(jnp.where(logits == m, lane, N_EXP), axis=1, keepdims=True)
    masked = jnp.where(lane == i1, -jnp.inf, logits)
    m2 = jnp.max(masked, axis=1, keepdims=True)
    i2 = jnp.min(jnp.where(masked == m2, lane, N_EXP), axis=1, keepdims=True)
    v1 = jnp.sum(jnp.where(lane == i1, probs, 0.0), axis=1, keepdims=True)
    v2 = jnp.sum(jnp.where(lane == i2, probs, 0.0), axis=1, keepdims=True)
    wts_ref[:, 0:1] = v1
    wts_ref[:, 1:2] = v2
    I1 = (lane == i1).astype(f32)
    I2 = (lane == i2).astype(f32)
    counts = jnp.sum(I1 + I2, axis=0, keepdims=True)
    nt = jnp.floor((counts + (TILE - 1)) * (1.0 / TILE))
    er = lax.broadcasted_iota(jnp.int32, (N_EXP, N_EXP), 0)
    ec = lax.broadcasted_iota(jnp.int32, (N_EXP, N_EXP), 1)
    strict = (er < ec).astype(f32)
    tiles_before = jnp.dot(nt, strict, preferred_element_type=f32)
    start = tiles_before * float(TILE)
    tiles_incl = tiles_before + nt
    ident = (er == ec).astype(f32)
    ti_col = jnp.sum(ident * tiles_incl, axis=1, keepdims=True)
    lane32 = lax.broadcasted_iota(jnp.int32, (1, 32), 1).astype(f32)
    te = jnp.sum((ti_col <= lane32).astype(f32), axis=0, keepdims=True)
    te = jnp.minimum(te, float(N_EXP - 1))
    used = tiles_incl[:, N_EXP - 1:N_EXP]
    meta_ref[...] = jnp.where(lane32 == 31.0, used, te).astype(jnp.int32)
    L = (lax.broadcasted_iota(jnp.int32, (128, 128), 0)
         >= lax.broadcasted_iota(jnp.int32, (128, 128), 1)).astype(f32)
    lane8 = lax.broadcasted_iota(jnp.int32, (128, N_EXP), 1)
    carry = jnp.zeros((1, N_EXP), f32)
    for blk in range(32):
        k, rb = divmod(blk, 16)
        r0 = rb * 128
        ei = (i1 if k == 0 else i2)[r0:r0 + 128, :]
        Ic = (lane8 == ei).astype(f32)
        incl = jnp.dot(L, Ic, preferred_element_type=f32)
        rank = carry + incl - Ic
        posb = jnp.sum(Ic * (start + rank), axis=1, keepdims=True)
        pos_ref[r0:r0 + 128, k:k + 1] = posb.astype(jnp.int32)
        carry = carry + incl[127:128, :]


def _router_call(x_flat, router_W, router_b):
    return pl.pallas_call(
        _router_body,
        out_shape=(
            jax.ShapeDtypeStruct((N_TOK, TOPK), jnp.int32),
            jax.ShapeDtypeStruct((N_TOK, TOPK), jnp.float32),
            jax.ShapeDtypeStruct((1, 32), jnp.int32),
            jax.ShapeDtypeStruct((1, 1), jnp.float32),
        ),
    )(x_flat, router_W, router_b.reshape(1, N_EXP))


def _sc_scatter_body(x_hbm, pos_hbm, xg_hbm, xbuf, idxbuf, sem):
    wid = lax.axis_index("s") * 2 + lax.axis_index("c")
    k = wid // 16
    blk = wid % 16
    for sub in range(2):
        tb = blk * 128 + sub * CH
        pltpu.sync_copy(x_hbm.at[pl.ds(tb, CH)], xbuf)
        pltpu.sync_copy(pos_hbm.at[pl.ds(k * N_TOK + tb, CH)], idxbuf)
        pltpu.async_copy(xbuf, xg_hbm.at[idxbuf], sem).wait()


def _sc_scatter(x_flat, pos_flat):
    mesh = plsc.VectorSubcoreMesh(core_axis_name="c", subcore_axis_name="s")
    fn = pl.kernel(
        _sc_scatter_body, mesh=mesh,
        out_type=jax.ShapeDtypeStruct((PROWS, C_DIM), jnp.float32),
        scratch_types=[pltpu.VMEM((CH, C_DIM), jnp.float32),
                       pltpu.VMEM((CH,), jnp.int32),
                       pltpu.SemaphoreType.DMA],
    )
    return fn(x_flat, pos_flat)


def _ffn_body(meta_ref, xg_ref, w1_ref, b1_ref, w2_ref, b2_ref, out_ref):
    t = pl.program_id(0)
    used = meta_ref[31]

    @pl.when(t < used)
    def _():
        xb = xg_ref[...].astype(jnp.bfloat16)
        h = jnp.dot(xb, w1_ref[0],
                    preferred_element_type=jnp.float32) + b1_ref[0]
        h = 0.5 * h * (1.0 + lax.erf(h * 0.7071067811865476))
        contrib = jnp.dot(h.astype(jnp.bfloat16), w2_ref[0],
                          preferred_element_type=jnp.float32)
        out_ref[...] = contrib + b2_ref[0]


def _ffn_call(meta, xg, W1, b1, W2, b2):
    grid_spec = pltpu.PrefetchScalarGridSpec(
        num_scalar_prefetch=1,
        grid=(MAXT,),
        in_specs=[
            pl.BlockSpec((TILE, C_DIM), lambda t, m: (t, 0)),
            pl.BlockSpec((1, C_DIM, F_DIM), lambda t, m: (m[t], 0, 0)),
            pl.BlockSpec((1, 1, F_DIM), lambda t, m: (m[t], 0, 0)),
            pl.BlockSpec((1, F_DIM, C_DIM), lambda t, m: (m[t], 0, 0)),
            pl.BlockSpec((1, 1, C_DIM), lambda t, m: (m[t], 0, 0)),
        ],
        out_specs=pl.BlockSpec((TILE, C_DIM), lambda t, m: (t, 0)),
    )
    return pl.pallas_call(
        _ffn_body, grid_spec=grid_spec,
        out_shape=jax.ShapeDtypeStruct((PROWS, C_DIM), jnp.float32),
    )(meta, xg, W1, b1, W2, b2)


def _sc_gather_body(o_hbm, pos_hbm, g0_hbm, g1_hbm, rbuf, idxbuf, sem):
    wid = lax.axis_index("s") * 2 + lax.axis_index("c")
    tb = wid * CH
    for k in range(2):
        pltpu.sync_copy(pos_hbm.at[pl.ds(k * N_TOK + tb, CH)], idxbuf)
        pltpu.async_copy(o_hbm.at[idxbuf], rbuf, sem).wait()
        g = g0_hbm if k == 0 else g1_hbm
        pltpu.sync_copy(rbuf, g.at[pl.ds(tb, CH)])


def _sc_gather(o, pos_flat):
    mesh = plsc.VectorSubcoreMesh(core_axis_name="c", subcore_axis_name="s")
    fn = pl.kernel(
        _sc_gather_body, mesh=mesh,
        out_type=(jax.ShapeDtypeStruct((N_TOK, C_DIM), jnp.float32),
                  jax.ShapeDtypeStruct((N_TOK, C_DIM), jnp.float32)),
        scratch_types=[pltpu.VMEM((CH, C_DIM), jnp.float32),
                       pltpu.VMEM((CH,), jnp.int32),
                       pltpu.SemaphoreType.DMA],
    )
    return fn(o, pos_flat)


def _combine_body(g0_ref, g1_ref, w_ref, out_ref):
    w = w_ref[...]
    out_ref[...] = g0_ref[...] * w[:, 0:1] + g1_ref[...] * w[:, 1:2]


def _combine_call(g0, g1, wts):
    RB = 512
    return pl.pallas_call(
        _combine_body,
        grid=(N_TOK // RB,),
        in_specs=[pl.BlockSpec((RB, C_DIM), lambda i: (i, 0)),
                  pl.BlockSpec((RB, C_DIM), lambda i: (i, 0)),
                  pl.BlockSpec((RB, TOPK), lambda i: (i, 0))],
        out_specs=pl.BlockSpec((RB, C_DIM), lambda i: (i, 0)),
        out_shape=jax.ShapeDtypeStruct((N_TOK, C_DIM), jnp.float32),
    )(g0, g1, wts)


def kernel(x, router_W, router_b, W1, b1, W2, b2):
    B, T, C = x.shape
    x_flat = x.reshape(T, C)
    pos, wts, meta, bal = _router_call(x_flat, router_W, router_b)
    pos_flat = pos.T.reshape(TOPK * N_TOK)
    xg = _sc_scatter(x_flat, pos_flat)
    o = _ffn_call(meta.reshape(32), xg,
                  W1.astype(jnp.bfloat16), b1.reshape(N_EXP, 1, F_DIM),
                  W2.astype(jnp.bfloat16), b2.reshape(N_EXP, 1, C_DIM))
    g0, g1 = _sc_gather(o, pos_flat)
    out = _combine_call(g0, g1, wts)
    return out.reshape(B, T, C), bal.reshape(())


# speedup vs baseline: 3.5568x; 1.1627x over previous
import jax
import jax.numpy as jnp
from jax import lax
from jax.experimental import pallas as pl
from jax.experimental.pallas import tpu as pltpu
from jax.experimental.pallas import tpu_sc as plsc

N_TOK = 2048
C_DIM = 1024
N_EXP = 8
F_DIM = 4096
TOPK = 2
TILE = 256
MAXT = 23
PROWS = MAXT * TILE
FBLK = 1024
NFB = F_DIM // FBLK
CH = 64


def _router_body(x_ref, w_ref, b_ref, pos_ref, wts_ref, meta_ref, bal_ref):
    f32 = jnp.float32
    xv = x_ref[...]
    logits = jnp.dot(xv, w_ref[...], preferred_element_type=f32) + b_ref[...]
    m = jnp.max(logits, axis=1, keepdims=True)
    ex = jnp.exp(logits - m)
    probs = ex / jnp.sum(ex, axis=1, keepdims=True)
    mean_p = jnp.sum(probs, axis=0, keepdims=True) * (1.0 / N_TOK)
    bal_ref[...] = jnp.sum(mean_p * mean_p, axis=1, keepdims=True)
    lane = lax.broadcasted_iota(jnp.int32, (N_TOK, N_EXP), 1)
    i1 = jnp.min(jnp.where(logits == m, lane, N_EXP), axis=1, keepdims=True)
    masked = jnp.where(lane == i1, -jnp.inf, logits)
    m2 = jnp.max(masked, axis=1, keepdims=True)
    i2 = jnp.min(jnp.where(masked == m2, lane, N_EXP), axis=1, keepdims=True)
    v1 = jnp.sum(jnp.where(lane == i1, probs, 0.0), axis=1, keepdims=True)
    v2 = jnp.sum(jnp.where(lane == i2, probs, 0.0), axis=1, keepdims=True)
    wts_ref[:, 0:1] = v1
    wts_ref[:, 1:2] = v2
    I1 = (lane == i1).astype(f32)
    I2 = (lane == i2).astype(f32)
    counts = jnp.sum(I1 + I2, axis=0, keepdims=True)
    nt = jnp.floor((counts + (TILE - 1)) * (1.0 / TILE))
    er = lax.broadcasted_iota(jnp.int32, (N_EXP, N_EXP), 0)
    ec = lax.broadcasted_iota(jnp.int32, (N_EXP, N_EXP), 1)
    strict = (er < ec).astype(f32)
    tiles_before = jnp.dot(nt, strict, preferred_element_type=f32)
    start = tiles_before * float(TILE)
    tiles_incl = tiles_before + nt
    ident = (er == ec).astype(f32)
    ti_col = jnp.sum(ident * tiles_incl, axis=1, keepdims=True)
    lane32 = lax.broadcasted_iota(jnp.int32, (1, 32), 1).astype(f32)
    te = jnp.sum((ti_col <= lane32).astype(f32), axis=0, keepdims=True)
    te = jnp.minimum(te, float(N_EXP - 1))
    used = tiles_incl[:, N_EXP - 1:N_EXP]
    meta_ref[...] = jnp.where(lane32 == 31.0, used, te).astype(jnp.int32)
    L = (lax.broadcasted_iota(jnp.int32, (128, 128), 0)
         >= lax.broadcasted_iota(jnp.int32, (128, 128), 1)).astype(f32)
    lane8 = lax.broadcasted_iota(jnp.int32, (128, N_EXP), 1)
    carry = jnp.zeros((1, N_EXP), f32)
    for blk in range(32):
        k, rb = divmod(blk, 16)
        r0 = rb * 128
        ei = (i1 if k == 0 else i2)[r0:r0 + 128, :]
        Ic = (lane8 == ei).astype(f32)
        incl = jnp.dot(L, Ic, preferred_element_type=f32)
        rank = carry + incl - Ic
        posb = jnp.sum(Ic * (start + rank), axis=1, keepdims=True)
        pos_ref[r0:r0 + 128, k:k + 1] = posb.astype(jnp.int32)
        carry = carry + incl[127:128, :]


def _router_call(x_flat, router_W, router_b):
    return pl.pallas_call(
        _router_body,
        out_shape=(
            jax.ShapeDtypeStruct((N_TOK, TOPK), jnp.int32),
            jax.ShapeDtypeStruct((N_TOK, TOPK), jnp.float32),
            jax.ShapeDtypeStruct((1, 32), jnp.int32),
            jax.ShapeDtypeStruct((1, 1), jnp.float32),
        ),
    )(x_flat, router_W, router_b.reshape(1, N_EXP))


def _sc_scatter_body(x_hbm, pos_hbm, xg_hbm, xbuf, idxbuf, sem):
    wid = lax.axis_index("s") * 2 + lax.axis_index("c")
    k = wid // 16
    blk = wid % 16
    for sub in range(2):
        tb = blk * 128 + sub * CH
        pltpu.sync_copy(x_hbm.at[pl.ds(tb, CH)], xbuf)
        pltpu.sync_copy(pos_hbm.at[pl.ds(k * N_TOK + tb, CH)], idxbuf)
        pltpu.async_copy(xbuf, xg_hbm.at[idxbuf], sem).wait()


def _sc_scatter(x_flat, pos_flat):
    mesh = plsc.VectorSubcoreMesh(core_axis_name="c", subcore_axis_name="s")
    fn = pl.kernel(
        _sc_scatter_body, mesh=mesh,
        out_type=jax.ShapeDtypeStruct((PROWS, C_DIM), jnp.float32),
        scratch_types=[pltpu.VMEM((CH, C_DIM), jnp.float32),
                       pltpu.VMEM((CH,), jnp.int32),
                       pltpu.SemaphoreType.DMA],
    )
    return fn(x_flat, pos_flat)


def _ffn1_body(meta_ref, xg_ref, w1_ref, b1_ref, h_ref):
    t = pl.program_id(0)
    used = meta_ref[31]

    @pl.when(t < used)
    def _():
        xb = xg_ref[...].astype(jnp.bfloat16)
        h = jnp.dot(xb, w1_ref[0].astype(jnp.bfloat16),
                    preferred_element_type=jnp.float32) + b1_ref[0]
        h = 0.5 * h * (1.0 + lax.erf(h * 0.7071067811865476))
        h_ref[...] = h.astype(jnp.bfloat16)


def _ffn2_body(meta_ref, h_ref, w2_ref, b2_ref, out_ref):
    t = pl.program_id(0)
    used = meta_ref[31]

    @pl.when(t < used)
    def _():
        contrib = jnp.dot(h_ref[...], w2_ref[0].astype(jnp.bfloat16),
                          preferred_element_type=jnp.float32)
        out_ref[...] = contrib + b2_ref[0]


def _ffn_call(meta, xg, W1, b1, W2, b2):
    grid_spec1 = pltpu.PrefetchScalarGridSpec(
        num_scalar_prefetch=1,
        grid=(MAXT,),
        in_specs=[
            pl.BlockSpec((TILE, C_DIM), lambda t, m: (t, 0)),
            pl.BlockSpec((1, C_DIM, F_DIM), lambda t, m: (m[t], 0, 0)),
            pl.BlockSpec((1, 1, F_DIM), lambda t, m: (m[t], 0, 0)),
        ],
        out_specs=pl.BlockSpec((TILE, F_DIM), lambda t, m: (t, 0)),
    )
    h = pl.pallas_call(
        _ffn1_body, grid_spec=grid_spec1,
        out_shape=jax.ShapeDtypeStruct((PROWS, F_DIM), jnp.bfloat16),
        compiler_params=pltpu.CompilerParams(vmem_limit_bytes=60 * 1024 * 1024),
    )(meta, xg, W1, b1)
    grid_spec2 = pltpu.PrefetchScalarGridSpec(
        num_scalar_prefetch=1,
        grid=(MAXT,),
        in_specs=[
            pl.BlockSpec((TILE, F_DIM), lambda t, m: (t, 0)),
            pl.BlockSpec((1, F_DIM, C_DIM), lambda t, m: (m[t], 0, 0)),
            pl.BlockSpec((1, 1, C_DIM), lambda t, m: (m[t], 0, 0)),
        ],
        out_specs=pl.BlockSpec((TILE, C_DIM), lambda t, m: (t, 0)),
    )
    return pl.pallas_call(
        _ffn2_body, grid_spec=grid_spec2,
        out_shape=jax.ShapeDtypeStruct((PROWS, C_DIM), jnp.float32),
        compiler_params=pltpu.CompilerParams(vmem_limit_bytes=60 * 1024 * 1024),
    )(meta, h, W2, b2)


def _sc_gather_body(o_hbm, pos_hbm, g0_hbm, g1_hbm, rbuf, idxbuf, sem):
    wid = lax.axis_index("s") * 2 + lax.axis_index("c")
    tb = wid * CH
    for k in range(2):
        pltpu.sync_copy(pos_hbm.at[pl.ds(k * N_TOK + tb, CH)], idxbuf)
        pltpu.async_copy(o_hbm.at[idxbuf], rbuf, sem).wait()
        g = g0_hbm if k == 0 else g1_hbm
        pltpu.sync_copy(rbuf, g.at[pl.ds(tb, CH)])


def _sc_gather(o, pos_flat):
    mesh = plsc.VectorSubcoreMesh(core_axis_name="c", subcore_axis_name="s")
    fn = pl.kernel(
        _sc_gather_body, mesh=mesh,
        out_type=(jax.ShapeDtypeStruct((N_TOK, C_DIM), jnp.float32),
                  jax.ShapeDtypeStruct((N_TOK, C_DIM), jnp.float32)),
        scratch_types=[pltpu.VMEM((CH, C_DIM), jnp.float32),
                       pltpu.VMEM((CH,), jnp.int32),
                       pltpu.SemaphoreType.DMA],
    )
    return fn(o, pos_flat)


def _combine_body(g0_ref, g1_ref, w_ref, out_ref):
    w = w_ref[...]
    out_ref[...] = g0_ref[...] * w[:, 0:1] + g1_ref[...] * w[:, 1:2]


def _combine_call(g0, g1, wts):
    RB = 512
    return pl.pallas_call(
        _combine_body,
        grid=(N_TOK // RB,),
        in_specs=[pl.BlockSpec((RB, C_DIM), lambda i: (i, 0)),
                  pl.BlockSpec((RB, C_DIM), lambda i: (i, 0)),
                  pl.BlockSpec((RB, TOPK), lambda i: (i, 0))],
        out_specs=pl.BlockSpec((RB, C_DIM), lambda i: (i, 0)),
        out_shape=jax.ShapeDtypeStruct((N_TOK, C_DIM), jnp.float32),
    )(g0, g1, wts)


def kernel(x, router_W, router_b, W1, b1, W2, b2):
    B, T, C = x.shape
    x_flat = x.reshape(T, C)
    pos, wts, meta, bal = _router_call(x_flat, router_W, router_b)
    pos_flat = pos.T.reshape(TOPK * N_TOK)
    xg = _sc_scatter(x_flat, pos_flat)
    o = _ffn_call(meta.reshape(32), xg,
                  W1, b1.reshape(N_EXP, 1, F_DIM),
                  W2, b2.reshape(N_EXP, 1, C_DIM))
    g0, g1 = _sc_gather(o, pos_flat)
    out = _combine_call(g0, g1, wts)
    return out.reshape(B, T, C), bal.reshape(())
